# Initial kernel scaffold; baseline (speedup 1.0000x reference)
import jax
import jax.numpy as jnp
from jax.experimental import pallas as pl


def kernel(states, enc_w1, enc_b1, enc_w2, enc_b2, enc_w3, enc_b3, proj_w, proj_b, flow_wh, flow_bh, flow_wmu, flow_bmu, flow_wa, flow_ba):
    raise NotImplementedError("write your pallas kernel here")


# speedup vs baseline: 74.2430x; 74.2430x over previous
import functools

import numpy as np

import jax
import jax.numpy as jnp
from jax.experimental import pallas as pl
from jax.experimental.pallas import tpu as pltpu

_STATE_DIM = 4
_CONCEPT_DIM = 8
_FLOW_DIM = 3
_H1, _H2 = 64, 32
_FLOW_LAYERS = 10
_FLOW_HID = 32
_TWO_PI = 6.283185307179586

_BLK = 8192
_CORES = 2


def _made_mask_constants():
    d_in = np.arange(1, _FLOW_DIM + 1)
    d_hid = (np.arange(_FLOW_HID) % (_FLOW_DIM - 1)) + 1
    m_h = (d_hid[None, :] >= d_in[:, None]).astype(np.float32)
    m_o = (d_in[None, :] > d_hid[:, None]).astype(np.float32)
    return m_h, m_o


_M_H, _M_O = _made_mask_constants()


def _body(xt_ref, w1t_ref, b1_ref, w2t_ref, b2_ref, w3t_ref, b3_ref,
          wh_ref, bh_ref, wma_ref, bma_ref, ot_ref):
    f32 = jnp.float32
    x = xt_ref[...]
    h = jnp.dot(w1t_ref[...], x, preferred_element_type=f32) + b1_ref[...]
    h = jnp.maximum(h, 0.0)
    h = jnp.dot(w2t_ref[...], h, preferred_element_type=f32) + b2_ref[...]
    h = jnp.maximum(h, 0.0)
    y = jnp.dot(w3t_ref[...], h, preferred_element_type=f32) + b3_ref[...]
    for l in range(_FLOW_LAYERS):
        whl = wh_ref[l]
        a, b = (0, 1) if l % 2 == 0 else (1, 2)
        hp = (whl[:, a:a + 1] * y[a:a + 1, :]
              + whl[:, b:b + 1] * y[b:b + 1, :] + bh_ref[l])
        hh = jnp.maximum(hp, 0.0)
        ma = jnp.dot(wma_ref[l], hh, preferred_element_type=f32) + bma_ref[l]
        mu = ma[0:3, :]
        alpha = jnp.tanh(ma[8:11, :])
        y = (y - mu) * jnp.exp(-alpha)
    u = jax.nn.sigmoid(y)
    row = jax.lax.broadcasted_iota(jnp.int32, (_FLOW_DIM, 1), 0)
    scale = jnp.where(row == 0, _TWO_PI, 1.0).astype(f32)
    ot_ref[...] = u * scale


@functools.partial(jax.jit, static_argnames=())
def kernel(states, enc_w1, enc_b1, enc_w2, enc_b2, enc_w3, enc_b3,
           proj_w, proj_b, flow_wh, flow_bh, flow_wmu, flow_bmu,
           flow_wa, flow_ba):
    f32 = jnp.float32
    B = states.shape[0]

    m_h = jnp.asarray(_M_H)
    m_o = jnp.asarray(_M_O)
    odd = (jnp.arange(_FLOW_LAYERS) % 2).astype(bool)

    wh_m = flow_wh * m_h[None]
    wmu_m = flow_wmu * m_o[None]
    wa_m = flow_wa * m_o[None]

    wh_k = jnp.where(odd[:, None, None], wh_m[:, ::-1, :], wh_m)
    wmu_k = jnp.where(odd[:, None, None], wmu_m[:, :, ::-1], wmu_m)
    wa_k = jnp.where(odd[:, None, None], wa_m[:, :, ::-1], wa_m)
    bmu_k = jnp.where(odd[:, None], flow_bmu[:, ::-1], flow_bmu)
    ba_k = jnp.where(odd[:, None], flow_ba[:, ::-1], flow_ba)

    w1t = enc_w1.T
    w2t = enc_w2.T
    w3p = enc_w3 @ proj_w
    b3p = enc_b3 @ proj_w + proj_b
    w3t = w3p.T

    wh_t = wh_k.transpose(0, 2, 1)
    z5 = jnp.zeros((_FLOW_LAYERS, 5, _FLOW_HID), f32)
    wma = jnp.concatenate(
        [wmu_k.transpose(0, 2, 1), z5, wa_k.transpose(0, 2, 1), z5],
        axis=1)
    z5b = jnp.zeros((_FLOW_LAYERS, 5), f32)
    bma = jnp.concatenate([bmu_k, z5b, ba_k, z5b], axis=1)

    b1c = enc_b1.reshape(_H1, 1)
    b2c = enc_b2.reshape(_H2, 1)
    b3c = b3p.reshape(_FLOW_DIM, 1)
    bhc = flow_bh.reshape(_FLOW_LAYERS, _FLOW_HID, 1)
    bmac = bma.reshape(_FLOW_LAYERS, 16, 1)

    nb2 = -(-B // (_CORES * _BLK))
    Bp = _CORES * nb2 * _BLK
    xt = jnp.zeros((_STATE_DIM, Bp), f32).at[:, :B].set(states.T)

    def _full(a):
        return pl.BlockSpec(a.shape, lambda i, j: (0,) * a.ndim)

    weights = (w1t, b1c, w2t, b2c, w3t, b3c, wh_t, bhc, wma, bmac)

    out_t = pl.pallas_call(
        _body,
        out_shape=jax.ShapeDtypeStruct((_FLOW_DIM, Bp), f32),
        grid=(_CORES, nb2),
        in_specs=[
            pl.BlockSpec((_STATE_DIM, _BLK),
                         lambda i, j: (0, i * nb2 + j)),
        ] + [_full(a) for a in weights],
        out_specs=pl.BlockSpec((_FLOW_DIM, _BLK),
                               lambda i, j: (0, i * nb2 + j)),
        compiler_params=pltpu.CompilerParams(
            dimension_semantics=("parallel", "arbitrary"),
        ),
        name="gsi_fused",
    )(xt, *weights)

    return out_t[:, :B].T


# speedup vs baseline: 87.5317x; 1.1790x over previous
import functools

import numpy as np

import jax
import jax.numpy as jnp
from jax.experimental import pallas as pl
from jax.experimental.pallas import tpu as pltpu

_STATE_DIM = 4
_CONCEPT_DIM = 8
_FLOW_DIM = 3
_H1, _H2 = 64, 32
_FLOW_LAYERS = 10
_FLOW_HID = 32
_TWO_PI = 6.283185307179586

_BLK = 8192
_CORES = 2


def _made_mask_constants():
    d_in = np.arange(1, _FLOW_DIM + 1)
    d_hid = (np.arange(_FLOW_HID) % (_FLOW_DIM - 1)) + 1
    m_h = (d_hid[None, :] >= d_in[:, None]).astype(np.float32)
    m_o = (d_in[None, :] > d_hid[:, None]).astype(np.float32)
    return m_h, m_o


_M_H, _M_O = _made_mask_constants()


def _body(xt_ref, w1t_ref, b1_ref, w2t_ref, b2_ref, w3t_ref, b3_ref,
          wh_ref, bh_ref, wma_ref, bma_ref, ot_ref):
    cols = slice(None)
    f32 = jnp.float32
    bf16 = jnp.bfloat16
    x = xt_ref[:, cols]
    h = jnp.dot(w1t_ref[...], x, preferred_element_type=f32) + b1_ref[...]
    h = jnp.maximum(h, 0.0).astype(bf16)
    h = jnp.dot(w2t_ref[...], h, preferred_element_type=f32) + b2_ref[...]
    h = jnp.maximum(h, 0.0).astype(bf16)
    y = jnp.dot(w3t_ref[...], h, preferred_element_type=f32) + b3_ref[...]
    for l in range(_FLOW_LAYERS):
        whl = wh_ref[l]
        bhl = bh_ref[l]
        a, b = (0, 1) if l % 2 == 0 else (1, 2)
        ya = y[a:a + 1, :].astype(bf16)
        yb = y[b:b + 1, :].astype(bf16)
        hp1 = whl[0:16, a:a + 1] * ya + bhl[0:16]
        hp2 = (whl[16:32, a:a + 1] * ya
               + (whl[16:32, b:b + 1] * yb + bhl[16:32]))
        hh = jnp.concatenate(
            [jnp.maximum(hp1, jnp.bfloat16(0)),
             jnp.maximum(hp2, jnp.bfloat16(0))], axis=0)
        ma = jnp.dot(wma_ref[l], hh, preferred_element_type=f32) + bma_ref[l]
        mu = ma[0:3, :]
        alpha = jnp.tanh(ma[4:7, :])
        y = (y - mu) * jnp.exp(-alpha)
    u = jax.nn.sigmoid(y)
    row = jax.lax.broadcasted_iota(jnp.int32, (_FLOW_DIM, 1), 0)
    scale = jnp.where(row == 0, _TWO_PI, 1.0).astype(f32)
    ot_ref[:, cols] = u * scale


@functools.partial(jax.jit, static_argnames=())
def kernel(states, enc_w1, enc_b1, enc_w2, enc_b2, enc_w3, enc_b3,
           proj_w, proj_b, flow_wh, flow_bh, flow_wmu, flow_bmu,
           flow_wa, flow_ba):
    f32 = jnp.float32
    B = states.shape[0]

    m_h = jnp.asarray(_M_H)
    m_o = jnp.asarray(_M_O)
    odd = (jnp.arange(_FLOW_LAYERS) % 2).astype(bool)

    wh_m = flow_wh * m_h[None]
    wmu_m = flow_wmu * m_o[None]
    wa_m = flow_wa * m_o[None]

    wh_k = jnp.where(odd[:, None, None], wh_m[:, ::-1, :], wh_m)
    wmu_k = jnp.where(odd[:, None, None], wmu_m[:, :, ::-1], wmu_m)
    wa_k = jnp.where(odd[:, None, None], wa_m[:, :, ::-1], wa_m)
    bmu_k = jnp.where(odd[:, None], flow_bmu[:, ::-1], flow_bmu)
    ba_k = jnp.where(odd[:, None], flow_ba[:, ::-1], flow_ba)

    w1t = enc_w1.T
    w2t = enc_w2.T
    w3p = enc_w3 @ proj_w
    b3p = enc_b3 @ proj_w + proj_b
    w3t = w3p.T

    hperm = np.concatenate([np.arange(0, _FLOW_HID, 2),
                            np.arange(1, _FLOW_HID, 2)])
    wh_t = wh_k.transpose(0, 2, 1)[:, hperm, :]
    z1 = jnp.zeros((_FLOW_LAYERS, 1, _FLOW_HID), f32)
    wma = jnp.concatenate(
        [wmu_k.transpose(0, 2, 1), z1, wa_k.transpose(0, 2, 1), z1],
        axis=1)[:, :, hperm]
    z1b = jnp.zeros((_FLOW_LAYERS, 1), f32)
    bma = jnp.concatenate([bmu_k, z1b, ba_k, z1b], axis=1)

    b1c = enc_b1.reshape(_H1, 1)
    b2c = enc_b2.reshape(_H2, 1)
    b3c = b3p.reshape(_FLOW_DIM, 1)
    bhc = flow_bh[:, hperm].reshape(_FLOW_LAYERS, _FLOW_HID, 1)
    bmac = bma.reshape(_FLOW_LAYERS, 8, 1)

    bf16 = jnp.bfloat16
    w1t = w1t.astype(bf16)
    w2t = w2t.astype(bf16)
    w3t = w3t.astype(bf16)
    wh_t = wh_t.astype(bf16)
    wma = wma.astype(bf16)
    bhc = bhc.astype(bf16)

    nb2 = -(-B // (_CORES * _BLK))
    Bp = _CORES * nb2 * _BLK
    xt = jnp.zeros((_STATE_DIM, Bp), bf16).at[:, :B].set(states.T.astype(bf16))

    def _full(a):
        return pl.BlockSpec(a.shape, lambda i, j: (0,) * a.ndim)

    weights = (w1t, b1c, w2t, b2c, w3t, b3c, wh_t, bhc, wma, bmac)

    out_t = pl.pallas_call(
        _body,
        out_shape=jax.ShapeDtypeStruct((_FLOW_DIM, Bp), f32),
        grid=(_CORES, nb2),
        in_specs=[
            pl.BlockSpec((_STATE_DIM, _BLK),
                         lambda i, j: (0, i * nb2 + j)),
        ] + [_full(a) for a in weights],
        out_specs=pl.BlockSpec((_FLOW_DIM, _BLK),
                               lambda i, j: (0, i * nb2 + j)),
        compiler_params=pltpu.CompilerParams(
            dimension_semantics=("parallel", "arbitrary"),
        ),
        name="gsi_fused",
    )(xt, *weights)

    return out_t[:, :B].T


# speedup vs baseline: 94.0841x; 1.0749x over previous
import functools

import numpy as np

import jax
import jax.numpy as jnp
from jax.experimental import pallas as pl
from jax.experimental.pallas import tpu as pltpu

_STATE_DIM = 4
_CONCEPT_DIM = 8
_FLOW_DIM = 3
_H1, _H2 = 64, 32
_FLOW_LAYERS = 10
_FLOW_HID = 32
_TWO_PI = 6.283185307179586

_BLK = 32768
_CORES = 2


def _made_mask_constants():
    d_in = np.arange(1, _FLOW_DIM + 1)
    d_hid = (np.arange(_FLOW_HID) % (_FLOW_DIM - 1)) + 1
    m_h = (d_hid[None, :] >= d_in[:, None]).astype(np.float32)
    m_o = (d_in[None, :] > d_hid[:, None]).astype(np.float32)
    return m_h, m_o


_M_H, _M_O = _made_mask_constants()


def _body(xt_ref, w1t_ref, b1_ref, w2t_ref, b2_ref, w3t_ref, b3_ref,
          wh_ref, bh_ref, wma_ref, bma_ref, ot_ref):
    cols = slice(None)
    f32 = jnp.float32
    bf16 = jnp.bfloat16
    x = xt_ref[:, cols]
    h = jnp.dot(w1t_ref[...], x, preferred_element_type=f32) + b1_ref[...]
    h = jnp.maximum(h, 0.0).astype(bf16)
    h = jnp.dot(w2t_ref[...], h, preferred_element_type=f32) + b2_ref[...]
    h = jnp.maximum(h, 0.0).astype(bf16)
    y = jnp.dot(w3t_ref[...], h, preferred_element_type=f32) + b3_ref[...]
    for l in range(_FLOW_LAYERS):
        whl = wh_ref[l]
        bhl = bh_ref[l]
        a, b = (0, 1) if l % 2 == 0 else (1, 2)
        ya = y[a:a + 1, :].astype(bf16)
        yb = y[b:b + 1, :].astype(bf16)
        hp1 = whl[0:16, a:a + 1] * ya + bhl[0:16]
        hp2 = (whl[16:32, a:a + 1] * ya
               + (whl[16:32, b:b + 1] * yb + bhl[16:32]))
        hh = jnp.concatenate(
            [jnp.maximum(hp1, jnp.bfloat16(0)),
             jnp.maximum(hp2, jnp.bfloat16(0))], axis=0)
        ma = jnp.dot(wma_ref[l], hh, preferred_element_type=f32) + bma_ref[l]
        mu = ma[0:3, :]
        alpha = jnp.tanh(ma[4:7, :])
        y = (y - mu) * jnp.exp(-alpha)
    u = jax.nn.sigmoid(y)
    row = jax.lax.broadcasted_iota(jnp.int32, (_FLOW_DIM, 1), 0)
    scale = jnp.where(row == 0, _TWO_PI, 1.0).astype(f32)
    ot_ref[:, cols] = u * scale


@functools.partial(jax.jit, static_argnames=())
def kernel(states, enc_w1, enc_b1, enc_w2, enc_b2, enc_w3, enc_b3,
           proj_w, proj_b, flow_wh, flow_bh, flow_wmu, flow_bmu,
           flow_wa, flow_ba):
    f32 = jnp.float32
    B = states.shape[0]

    m_h = jnp.asarray(_M_H)
    m_o = jnp.asarray(_M_O)
    odd = (jnp.arange(_FLOW_LAYERS) % 2).astype(bool)

    wh_m = flow_wh * m_h[None]
    wmu_m = flow_wmu * m_o[None]
    wa_m = flow_wa * m_o[None]

    wh_k = jnp.where(odd[:, None, None], wh_m[:, ::-1, :], wh_m)
    wmu_k = jnp.where(odd[:, None, None], wmu_m[:, :, ::-1], wmu_m)
    wa_k = jnp.where(odd[:, None, None], wa_m[:, :, ::-1], wa_m)
    bmu_k = jnp.where(odd[:, None], flow_bmu[:, ::-1], flow_bmu)
    ba_k = jnp.where(odd[:, None], flow_ba[:, ::-1], flow_ba)

    w1t = enc_w1.T
    w2t = enc_w2.T
    w3p = enc_w3 @ proj_w
    b3p = enc_b3 @ proj_w + proj_b
    w3t = w3p.T

    hperm = np.concatenate([np.arange(0, _FLOW_HID, 2),
                            np.arange(1, _FLOW_HID, 2)])
    wh_t = wh_k.transpose(0, 2, 1)[:, hperm, :]
    z1 = jnp.zeros((_FLOW_LAYERS, 1, _FLOW_HID), f32)
    wma = jnp.concatenate(
        [wmu_k.transpose(0, 2, 1), z1, wa_k.transpose(0, 2, 1), z1],
        axis=1)[:, :, hperm]
    z1b = jnp.zeros((_FLOW_LAYERS, 1), f32)
    bma = jnp.concatenate([bmu_k, z1b, ba_k, z1b], axis=1)

    b1c = enc_b1.reshape(_H1, 1)
    b2c = enc_b2.reshape(_H2, 1)
    b3c = b3p.reshape(_FLOW_DIM, 1)
    bhc = flow_bh[:, hperm].reshape(_FLOW_LAYERS, _FLOW_HID, 1)
    bmac = bma.reshape(_FLOW_LAYERS, 8, 1)

    bf16 = jnp.bfloat16
    w1t = w1t.astype(bf16)
    w2t = w2t.astype(bf16)
    w3t = w3t.astype(bf16)
    wh_t = wh_t.astype(bf16)
    wma = wma.astype(bf16)
    bhc = bhc.astype(bf16)

    nb2 = -(-B // (_CORES * _BLK))
    Bp = _CORES * nb2 * _BLK
    xt = jnp.zeros((_STATE_DIM, Bp), bf16).at[:, :B].set(states.T.astype(bf16))

    def _full(a):
        return pl.BlockSpec(a.shape, lambda i, j: (0,) * a.ndim)

    weights = (w1t, b1c, w2t, b2c, w3t, b3c, wh_t, bhc, wma, bmac)

    out_t = pl.pallas_call(
        _body,
        out_shape=jax.ShapeDtypeStruct((_FLOW_DIM, Bp), f32),
        grid=(_CORES, nb2),
        in_specs=[
            pl.BlockSpec((_STATE_DIM, _BLK),
                         lambda i, j: (0, i * nb2 + j)),
        ] + [_full(a) for a in weights],
        out_specs=pl.BlockSpec((_FLOW_DIM, _BLK),
                               lambda i, j: (0, i * nb2 + j)),
        compiler_params=pltpu.CompilerParams(
            dimension_semantics=("parallel", "arbitrary"),
        ),
        name="gsi_fused",
    )(xt, *weights)

    return out_t[:, :B].T


# speedup vs baseline: 96.6837x; 1.0276x over previous
import functools

import numpy as np

import jax
import jax.numpy as jnp
from jax.experimental import pallas as pl
from jax.experimental.pallas import tpu as pltpu

_STATE_DIM = 4
_CONCEPT_DIM = 8
_FLOW_DIM = 3
_H1, _H2 = 64, 32
_FLOW_LAYERS = 10
_FLOW_HID = 32
_TWO_PI = 6.283185307179586

_BLK = 32768
_CORES = 2


def _made_mask_constants():
    d_in = np.arange(1, _FLOW_DIM + 1)
    d_hid = (np.arange(_FLOW_HID) % (_FLOW_DIM - 1)) + 1
    m_h = (d_hid[None, :] >= d_in[:, None]).astype(np.float32)
    m_o = (d_in[None, :] > d_hid[:, None]).astype(np.float32)
    return m_h, m_o


_M_H, _M_O = _made_mask_constants()


def _body(xt_ref, w1t_ref, w2t_ref, w3t_ref, wh_ref, bh_ref, wma_ref,
          ot_ref):
    f32 = jnp.float32
    bf16 = jnp.bfloat16
    ones = jnp.ones((1, _BLK), bf16)
    x = xt_ref[...]
    h = jnp.dot(w1t_ref[...], x, preferred_element_type=f32)
    h = jnp.maximum(h.astype(bf16), jnp.bfloat16(0))
    h = jnp.concatenate([h, ones], axis=0)
    h = jnp.dot(w2t_ref[...], h, preferred_element_type=f32)
    h = jnp.maximum(h.astype(bf16), jnp.bfloat16(0))
    h = jnp.concatenate([h, ones], axis=0)
    y = jnp.dot(w3t_ref[...], h, preferred_element_type=f32)
    for l in range(_FLOW_LAYERS):
        whl = wh_ref[l]
        bhl = bh_ref[l]
        a, b = (0, 1) if l % 2 == 0 else (1, 2)
        y16 = y.astype(bf16)
        ya = jnp.broadcast_to(y16[a:a + 1, :], (16, _BLK))
        yb = jnp.broadcast_to(y16[b:b + 1, :], (16, _BLK))
        hp1 = whl[0:16, a:a + 1] * ya + bhl[0:16]
        hp2 = (whl[16:32, a:a + 1] * ya
               + (whl[16:32, b:b + 1] * yb + bhl[16:32]))
        hh = jnp.concatenate(
            [jnp.maximum(hp1, jnp.bfloat16(0)),
             jnp.maximum(hp2, jnp.bfloat16(0)),
             ones], axis=0)
        ma = jnp.dot(wma_ref[l], hh, preferred_element_type=f32)
        mu = ma[0:3, :]
        alpha = jnp.tanh(ma[8:11, :])
        y = (y - mu) * jnp.exp(-alpha)
    u = jax.nn.sigmoid(y)
    row = jax.lax.broadcasted_iota(jnp.int32, (_FLOW_DIM, 1), 0)
    scale = jnp.where(row == 0, _TWO_PI, 1.0).astype(f32)
    ot_ref[...] = u * scale


@functools.partial(jax.jit, static_argnames=())
def kernel(states, enc_w1, enc_b1, enc_w2, enc_b2, enc_w3, enc_b3,
           proj_w, proj_b, flow_wh, flow_bh, flow_wmu, flow_bmu,
           flow_wa, flow_ba):
    f32 = jnp.float32
    B = states.shape[0]

    m_h = jnp.asarray(_M_H)
    m_o = jnp.asarray(_M_O)
    odd = (jnp.arange(_FLOW_LAYERS) % 2).astype(bool)

    wh_m = flow_wh * m_h[None]
    wmu_m = flow_wmu * m_o[None]
    wa_m = flow_wa * m_o[None]

    wh_k = jnp.where(odd[:, None, None], wh_m[:, ::-1, :], wh_m)
    wmu_k = jnp.where(odd[:, None, None], wmu_m[:, :, ::-1], wmu_m)
    wa_k = jnp.where(odd[:, None, None], wa_m[:, :, ::-1], wa_m)
    bmu_k = jnp.where(odd[:, None], flow_bmu[:, ::-1], flow_bmu)
    ba_k = jnp.where(odd[:, None], flow_ba[:, ::-1], flow_ba)

    w1t = enc_w1.T
    w2t = enc_w2.T
    w3p = enc_w3 @ proj_w
    b3p = enc_b3 @ proj_w + proj_b
    w3t = w3p.T

    hperm = np.concatenate([np.arange(0, _FLOW_HID, 2),
                            np.arange(1, _FLOW_HID, 2)])
    wh_t = wh_k.transpose(0, 2, 1)[:, hperm, :]
    z5 = jnp.zeros((_FLOW_LAYERS, 5, _FLOW_HID), f32)
    wma = jnp.concatenate(
        [wmu_k.transpose(0, 2, 1), z5, wa_k.transpose(0, 2, 1), z5],
        axis=1)[:, :, hperm]
    z5b = jnp.zeros((_FLOW_LAYERS, 5), f32)
    bma = jnp.concatenate([bmu_k, z5b, ba_k, z5b], axis=1)

    bhc = flow_bh[:, hperm].reshape(_FLOW_LAYERS, _FLOW_HID, 1)

    w1t = jnp.concatenate([w1t, enc_b1.reshape(_H1, 1)], axis=1)
    w2t = jnp.concatenate([w2t, enc_b2.reshape(_H2, 1)], axis=1)
    w3t = jnp.concatenate([w3t, b3p.reshape(_FLOW_DIM, 1)], axis=1)
    wma = jnp.concatenate([wma, bma.reshape(_FLOW_LAYERS, 16, 1)],
                          axis=2)

    bf16 = jnp.bfloat16
    w1t = w1t.astype(bf16)
    w2t = w2t.astype(bf16)
    w3t = w3t.astype(bf16)
    wh_t = wh_t.astype(bf16)
    wma = wma.astype(bf16)
    bhc = bhc.astype(bf16)

    nb2 = -(-B // (_CORES * _BLK))
    Bp = _CORES * nb2 * _BLK
    xt = (jnp.ones((_STATE_DIM + 1, Bp), bf16)
          .at[:_STATE_DIM, :B].set(states.T.astype(bf16))
          .at[:_STATE_DIM, B:].set(0))

    def _full(a):
        return pl.BlockSpec(a.shape, lambda i, j: (0,) * a.ndim)

    weights = (w1t, w2t, w3t, wh_t, bhc, wma)

    out_t = pl.pallas_call(
        _body,
        out_shape=jax.ShapeDtypeStruct((_FLOW_DIM, Bp), f32),
        grid=(_CORES, nb2),
        in_specs=[
            pl.BlockSpec((_STATE_DIM + 1, _BLK),
                         lambda i, j: (0, i * nb2 + j)),
        ] + [_full(a) for a in weights],
        out_specs=pl.BlockSpec((_FLOW_DIM, _BLK),
                               lambda i, j: (0, i * nb2 + j)),
        compiler_params=pltpu.CompilerParams(
            dimension_semantics=("parallel", "arbitrary"),
        ),
        name="gsi_fused",
    )(xt, *weights)

    return out_t[:, :B].T
